# baseline (speedup 1.0000x reference)
import functools
import math

import jax
import jax.numpy as jnp
from jax.experimental import pallas as pl
from jax.experimental.pallas import tpu as pltpu

_B = 8192
_IN = 128
_HID = 128
_OUT = 128
_NN = 12
_TB = 256
_RSQ = 1.0 / math.sqrt(128.0)


def _fused_kernel(c_ref, xt_ref, w1t_ref, b1t_ref, w2t_ref, b2t_ref,
                  zft_ref,
                  l1wst_ref, l1wat_ref, l1bt_ref,
                  l2wst_ref, l2wat_ref, l2bt_ref, o_ref):
    xt = xt_ref[...]
    h1 = jnp.maximum(w1t_ref[...] @ xt + b1t_ref[...], 0.0)
    hs = []
    for n in range(_NN):
        h1n = h1[n * _HID:(n + 1) * _HID, :]
        hs.append(w2t_ref[n * _HID:(n + 1) * _HID, :] @ h1n
                  + b2t_ref[:, n:n + 1])

    ones_row = jnp.ones((1, _HID), jnp.float32)
    bias1 = l1bt_ref[...] + zft_ref[...]

    def gnn_layer(hcur, wst, wat, bias, use_relu):
        gp = {}
        for n in range(_NN):
            for m in range(n, _NN):
                gp[(n, m)] = (ones_row @ (hcur[n] * hcur[m])) * _RSQ

        def score(n, m):
            return gp[(n, m) if n <= m else (m, n)]

        aggs = []
        for n in range(_NN):
            cs = [c_ref[n, m] for m in range(_NN)]
            mx = jnp.full((1, _TB), -jnp.inf, jnp.float32)
            for m in range(_NN):
                mx = jnp.where(cs[m] > 0, jnp.maximum(mx, score(n, m)), mx)
            mx = jnp.where(jnp.isfinite(mx), mx, 0.0)
            exs = []
            den = jnp.zeros((1, _TB), jnp.float32)
            for m in range(_NN):
                e = jnp.exp(score(n, m) - mx)
                exs.append(e)
                den = den + cs[m] * e
            inv = 1.0 / (den + 1e-9)
            acc = ((cs[0] * exs[0]) * inv) * hcur[0]
            for m in range(1, _NN):
                acc = acc + ((cs[m] * exs[m]) * inv) * hcur[m]
            o = wst @ hcur[n] + wat @ acc + bias
            if use_relu:
                o = jnp.maximum(o, 0.0)
            aggs.append(o)
        return aggs

    hs = gnn_layer(hs, l1wst_ref[...], l1wat_ref[...], bias1, True)
    hs = gnn_layer(hs, l2wst_ref[...], l2wat_ref[...], l2bt_ref[...], False)

    acc = hs[0]
    for n in range(1, _NN):
        acc = acc + hs[n]
    o_ref[...] = acc * (1.0 / _NN)


def _full(shape):
    zeros = (0,) * len(shape)
    return pl.BlockSpec(shape, lambda i, z=zeros: z)


@jax.jit
def _run(c, xt, w1t, b1t, w2t, b2t, zft, l1wst, l1wat, l1bt, l2wst, l2wat, l2bt):
    out_t = pl.pallas_call(
        _fused_kernel,
        grid=(_B // _TB,),
        in_specs=[
            pl.BlockSpec(memory_space=pltpu.SMEM),
            pl.BlockSpec((_IN, _TB), lambda i: (0, i)),
            _full((_NN * _HID, _IN)),
            _full((_NN * _HID, 1)),
            _full((_NN * _HID, _HID)),
            _full((_OUT, _NN)),
            _full((_HID, 1)),
            _full((_HID, _OUT)),
            _full((_HID, _OUT)),
            _full((_HID, 1)),
            _full((_OUT, _HID)),
            _full((_OUT, _HID)),
            _full((_OUT, 1)),
        ],
        out_specs=pl.BlockSpec((_OUT, _TB), lambda i: (0, i)),
        out_shape=jax.ShapeDtypeStruct((_OUT, _B), jnp.float32),
        compiler_params=pltpu.CompilerParams(
            dimension_semantics=("parallel",)),
    )(c, xt, w1t, b1t, w2t, b2t, zft, l1wst, l1wat, l1bt, l2wst, l2wat, l2bt)
    return out_t.T


def kernel(x, edge_index, z, params):
    src = edge_index[0]
    dst = edge_index[1]
    c = jnp.zeros((_NN, _NN), jnp.float32).at[dst, src].add(1.0)
    xt = x.T
    w1t = params["gauge_W1"].transpose(0, 2, 1).reshape(_NN * _HID, _IN)
    b1t = params["gauge_b1"].reshape(_NN * _HID, 1)
    w2t = params["gauge_W2"].transpose(0, 2, 1).reshape(_NN * _HID, _HID)
    b2t = params["gauge_b2"].T
    zft = (z @ params["z_W"] + params["z_b"]).reshape(_HID, 1)
    out_t = _run(
        c, xt, w1t, b1t, w2t, b2t, zft,
        params["l1_Ws"].T, params["l1_Wa"].T, params["l1_b"].reshape(_HID, 1),
        params["l2_Ws"].T, params["l2_Wa"].T, params["l2_b"].reshape(_OUT, 1),
    )
    return out_t


# speedup vs baseline: 1.0354x; 1.0354x over previous
import functools
import math

import jax
import jax.numpy as jnp
from jax.experimental import pallas as pl
from jax.experimental.pallas import tpu as pltpu

_B = 8192
_IN = 128
_HID = 128
_OUT = 128
_NN = 12
_TB = 256
_RSQ = 1.0 / math.sqrt(128.0)


def _fused_kernel(c_ref, xt_ref, w1t_ref, b1t_ref, w2t_ref, b2t_ref,
                  zft_ref,
                  l1wst_ref, l1wat_ref, l1bt_ref,
                  l2wst_ref, l2wat_ref, l2bt_ref, o_ref):
    x = xt_ref[...]
    h1 = jnp.maximum(
        jax.lax.dot_general(w1t_ref[...], x, (((1,), (1,)), ((), ())))
        + b1t_ref[...], 0.0)
    hs = []
    for n in range(_NN):
        h1n = h1[n * _HID:(n + 1) * _HID, :]
        hs.append(w2t_ref[n * _HID:(n + 1) * _HID, :] @ h1n
                  + b2t_ref[:, n:n + 1])

    ones_row = jnp.ones((1, _HID), jnp.float32)
    bias1 = l1bt_ref[...] + zft_ref[...]

    def gnn_layer(hcur, wst, wat, bias, use_relu):
        gp = {}
        for n in range(_NN):
            for m in range(n, _NN):
                gp[(n, m)] = (ones_row @ (hcur[n] * hcur[m])) * _RSQ

        def score(n, m):
            return gp[(n, m) if n <= m else (m, n)]

        aggs = []
        for n in range(_NN):
            cs = [c_ref[n, m] for m in range(_NN)]
            mx = jnp.full((1, _TB), -jnp.inf, jnp.float32)
            for m in range(_NN):
                mx = jnp.where(cs[m] > 0, jnp.maximum(mx, score(n, m)), mx)
            mx = jnp.where(jnp.isfinite(mx), mx, 0.0)
            exs = []
            den = jnp.zeros((1, _TB), jnp.float32)
            for m in range(_NN):
                e = jnp.exp(score(n, m) - mx)
                exs.append(e)
                den = den + cs[m] * e
            inv = 1.0 / (den + 1e-9)
            acc = ((cs[0] * exs[0]) * inv) * hcur[0]
            for m in range(1, _NN):
                acc = acc + ((cs[m] * exs[m]) * inv) * hcur[m]
            o = wst @ hcur[n] + wat @ acc + bias
            if use_relu:
                o = jnp.maximum(o, 0.0)
            aggs.append(o)
        return aggs

    hs = gnn_layer(hs, l1wst_ref[...], l1wat_ref[...], bias1, True)
    hs = gnn_layer(hs, l2wst_ref[...], l2wat_ref[...], l2bt_ref[...], False)

    acc = hs[0]
    for n in range(1, _NN):
        acc = acc + hs[n]
    o_ref[...] = (acc * (1.0 / _NN)).T


def _full(shape):
    zeros = (0,) * len(shape)
    return pl.BlockSpec(shape, lambda i, z=zeros: z)


@jax.jit
def _run(c, xt, w1t, b1t, w2t, b2t, zft, l1wst, l1wat, l1bt, l2wst, l2wat, l2bt):
    out_t = pl.pallas_call(
        _fused_kernel,
        grid=(_B // _TB,),
        in_specs=[
            pl.BlockSpec(memory_space=pltpu.SMEM),
            pl.BlockSpec((_TB, _IN), lambda i: (i, 0)),
            _full((_NN * _HID, _IN)),
            _full((_NN * _HID, 1)),
            _full((_NN * _HID, _HID)),
            _full((_OUT, _NN)),
            _full((_HID, 1)),
            _full((_HID, _OUT)),
            _full((_HID, _OUT)),
            _full((_HID, 1)),
            _full((_OUT, _HID)),
            _full((_OUT, _HID)),
            _full((_OUT, 1)),
        ],
        out_specs=pl.BlockSpec((_TB, _OUT), lambda i: (i, 0)),
        out_shape=jax.ShapeDtypeStruct((_B, _OUT), jnp.float32),
        compiler_params=pltpu.CompilerParams(
            dimension_semantics=("parallel",)),
    )(c, xt, w1t, b1t, w2t, b2t, zft, l1wst, l1wat, l1bt, l2wst, l2wat, l2bt)
    return out_t


def kernel(x, edge_index, z, params):
    src = edge_index[0]
    dst = edge_index[1]
    c = jnp.zeros((_NN, _NN), jnp.float32).at[dst, src].add(1.0)
    w1t = params["gauge_W1"].transpose(0, 2, 1).reshape(_NN * _HID, _IN)
    b1t = params["gauge_b1"].reshape(_NN * _HID, 1)
    w2t = params["gauge_W2"].transpose(0, 2, 1).reshape(_NN * _HID, _HID)
    b2t = params["gauge_b2"].T
    zft = (z @ params["z_W"] + params["z_b"]).reshape(_HID, 1)
    out_t = _run(
        c, x, w1t, b1t, w2t, b2t, zft,
        params["l1_Ws"].T, params["l1_Wa"].T, params["l1_b"].reshape(_HID, 1),
        params["l2_Ws"].T, params["l2_Wa"].T, params["l2_b"].reshape(_OUT, 1),
    )
    return out_t


# speedup vs baseline: 1.0582x; 1.0220x over previous
import functools
import math

import jax
import jax.numpy as jnp
from jax.experimental import pallas as pl
from jax.experimental.pallas import tpu as pltpu

_B = 8192
_IN = 128
_HID = 128
_OUT = 128
_NN = 12
_TB = 256
_RSQ = 1.0 / math.sqrt(128.0)


def _fused_kernel(c_ref, xt_ref, w1t_ref, b1t_ref, w2t_ref, b2t_ref,
                  zft_ref,
                  l1wst_ref, l1wat_ref, l1bt_ref,
                  l2wst_ref, l2wat_ref, l2bt_ref, o_ref):
    def _tn(w, h):
        return jax.lax.dot_general(w, h, (((0,), (0,)), ((), ())))

    x = xt_ref[...]
    hs = []
    h1 = []
    for n in range(_NN):
        w1n = w1t_ref[n * _IN:(n + 1) * _IN, :]
        h1.append(jnp.maximum(
            jax.lax.dot_general(w1n, x, (((0,), (1,)), ((), ())))
            + b1t_ref[n * _HID:(n + 1) * _HID, :], 0.0))
    for n in range(_NN):
        hs.append(_tn(w2t_ref[n * _HID:(n + 1) * _HID, :], h1[n])
                  + b2t_ref[:, n:n + 1])

    ones_row = jnp.ones((1, _HID), jnp.float32)
    bias1 = l1bt_ref[...] + zft_ref[...]

    def gnn_layer(hcur, wst, wat, bias, use_relu):
        gp = {}
        for n in range(_NN):
            for m in range(n, _NN):
                gp[(n, m)] = (ones_row @ (hcur[n] * hcur[m])) * _RSQ

        def score(n, m):
            return gp[(n, m) if n <= m else (m, n)]

        aggs = []
        for n in range(_NN):
            cs = [c_ref[n, m] for m in range(_NN)]
            mx = jnp.full((1, _TB), -jnp.inf, jnp.float32)
            for m in range(_NN):
                mx = jnp.where(cs[m] > 0, jnp.maximum(mx, score(n, m)), mx)
            mx = jnp.where(jnp.isfinite(mx), mx, 0.0)
            exs = []
            den = jnp.zeros((1, _TB), jnp.float32)
            for m in range(_NN):
                e = jnp.exp(score(n, m) - mx)
                exs.append(e)
                den = den + cs[m] * e
            inv = 1.0 / (den + 1e-9)
            acc = ((cs[0] * exs[0]) * inv) * hcur[0]
            for m in range(1, _NN):
                acc = acc + ((cs[m] * exs[m]) * inv) * hcur[m]
            o = _tn(wst, hcur[n]) + _tn(wat, acc) + bias
            if use_relu:
                o = jnp.maximum(o, 0.0)
            aggs.append(o)
        return aggs

    hs = gnn_layer(hs, l1wst_ref[...], l1wat_ref[...], bias1, True)
    hs = gnn_layer(hs, l2wst_ref[...], l2wat_ref[...], l2bt_ref[...], False)

    acc = hs[0]
    for n in range(1, _NN):
        acc = acc + hs[n]
    o_ref[...] = (acc * (1.0 / _NN)).T


def _full(shape):
    zeros = (0,) * len(shape)
    return pl.BlockSpec(shape, lambda i, z=zeros: z)


@jax.jit
def _run(c, xt, w1t, b1t, w2t, b2t, zft, l1wst, l1wat, l1bt, l2wst, l2wat, l2bt):
    out_t = pl.pallas_call(
        _fused_kernel,
        grid=(_B // _TB,),
        in_specs=[
            pl.BlockSpec(memory_space=pltpu.SMEM),
            pl.BlockSpec((_TB, _IN), lambda i: (i, 0)),
            _full((_NN * _HID, _IN)),
            _full((_NN * _HID, 1)),
            _full((_NN * _HID, _HID)),
            _full((_OUT, _NN)),
            _full((_HID, 1)),
            _full((_HID, _OUT)),
            _full((_HID, _OUT)),
            _full((_HID, 1)),
            _full((_OUT, _HID)),
            _full((_OUT, _HID)),
            _full((_OUT, 1)),
        ],
        out_specs=pl.BlockSpec((_TB, _OUT), lambda i: (i, 0)),
        out_shape=jax.ShapeDtypeStruct((_B, _OUT), jnp.float32),
        compiler_params=pltpu.CompilerParams(
            dimension_semantics=("parallel",)),
    )(c, xt, w1t, b1t, w2t, b2t, zft, l1wst, l1wat, l1bt, l2wst, l2wat, l2bt)
    return out_t


def kernel(x, edge_index, z, params):
    src = edge_index[0]
    dst = edge_index[1]
    c = jnp.zeros((_NN, _NN), jnp.float32).at[dst, src].add(1.0)
    w1t = params["gauge_W1"].reshape(_NN * _IN, _HID)
    b1t = params["gauge_b1"].reshape(_NN * _HID, 1)
    w2t = params["gauge_W2"].reshape(_NN * _HID, _OUT)
    b2t = params["gauge_b2"].T
    zft = (z @ params["z_W"] + params["z_b"]).reshape(_HID, 1)
    out_t = _run(
        c, x, w1t, b1t, w2t, b2t, zft,
        params["l1_Ws"], params["l1_Wa"], params["l1_b"].reshape(_HID, 1),
        params["l2_Ws"], params["l2_Wa"], params["l2_b"].reshape(_OUT, 1),
    )
    return out_t


# speedup vs baseline: 1.0964x; 1.0361x over previous
import functools
import math

import jax
import jax.numpy as jnp
from jax.experimental import pallas as pl
from jax.experimental.pallas import tpu as pltpu

_B = 8192
_IN = 128
_HID = 128
_OUT = 128
_NN = 12
_TB = 512
_RSQ = 1.0 / math.sqrt(128.0)


def _fused_kernel(c_ref, xt_ref, w1t_ref, b1t_ref, w2t_ref, b2t_ref,
                  zft_ref,
                  l1wst_ref, l1wat_ref, l1bt_ref,
                  l2wst_ref, l2wat_ref, l2bt_ref, o_ref):
    def _tn(w, h):
        return jax.lax.dot_general(w, h, (((0,), (0,)), ((), ())))

    x = xt_ref[...]
    hs = []
    h1 = []
    for n in range(_NN):
        w1n = w1t_ref[n * _IN:(n + 1) * _IN, :]
        h1.append(jnp.maximum(
            jax.lax.dot_general(w1n, x, (((0,), (1,)), ((), ())))
            + b1t_ref[n * _HID:(n + 1) * _HID, :], 0.0))
    for n in range(_NN):
        hs.append(_tn(w2t_ref[n * _HID:(n + 1) * _HID, :], h1[n])
                  + b2t_ref[:, n:n + 1])

    ones_row = jnp.ones((1, _HID), jnp.float32)
    bias1 = l1bt_ref[...] + zft_ref[...]

    def gnn_layer(hcur, wst, wat, bias, use_relu):
        gp = {}
        for n in range(_NN):
            for m in range(n, _NN):
                gp[(n, m)] = (ones_row @ (hcur[n] * hcur[m])) * _RSQ

        def score(n, m):
            return gp[(n, m) if n <= m else (m, n)]

        aggs = []
        for n in range(_NN):
            cs = [c_ref[n, m] for m in range(_NN)]
            mx = jnp.full((1, _TB), -jnp.inf, jnp.float32)
            for m in range(_NN):
                mx = jnp.where(cs[m] > 0, jnp.maximum(mx, score(n, m)), mx)
            mx = jnp.where(jnp.isfinite(mx), mx, 0.0)
            exs = []
            den = jnp.zeros((1, _TB), jnp.float32)
            for m in range(_NN):
                e = jnp.exp(score(n, m) - mx)
                exs.append(e)
                den = den + cs[m] * e
            inv = 1.0 / (den + 1e-9)
            acc = ((cs[0] * exs[0]) * inv) * hcur[0]
            for m in range(1, _NN):
                acc = acc + ((cs[m] * exs[m]) * inv) * hcur[m]
            o = _tn(wst, hcur[n]) + _tn(wat, acc) + bias
            if use_relu:
                o = jnp.maximum(o, 0.0)
            aggs.append(o)
        return aggs

    hs = gnn_layer(hs, l1wst_ref[...], l1wat_ref[...], bias1, True)
    hs = gnn_layer(hs, l2wst_ref[...], l2wat_ref[...], l2bt_ref[...], False)

    acc = hs[0]
    for n in range(1, _NN):
        acc = acc + hs[n]
    o_ref[...] = (acc * (1.0 / _NN)).T


def _full(shape):
    zeros = (0,) * len(shape)
    return pl.BlockSpec(shape, lambda i, z=zeros: z)


@jax.jit
def _run(c, xt, w1t, b1t, w2t, b2t, zft, l1wst, l1wat, l1bt, l2wst, l2wat, l2bt):
    out_t = pl.pallas_call(
        _fused_kernel,
        grid=(_B // _TB,),
        in_specs=[
            pl.BlockSpec(memory_space=pltpu.SMEM),
            pl.BlockSpec((_TB, _IN), lambda i: (i, 0)),
            _full((_NN * _HID, _IN)),
            _full((_NN * _HID, 1)),
            _full((_NN * _HID, _HID)),
            _full((_OUT, _NN)),
            _full((_HID, 1)),
            _full((_HID, _OUT)),
            _full((_HID, _OUT)),
            _full((_HID, 1)),
            _full((_OUT, _HID)),
            _full((_OUT, _HID)),
            _full((_OUT, 1)),
        ],
        out_specs=pl.BlockSpec((_TB, _OUT), lambda i: (i, 0)),
        out_shape=jax.ShapeDtypeStruct((_B, _OUT), jnp.float32),
        compiler_params=pltpu.CompilerParams(
            dimension_semantics=("parallel",)),
    )(c, xt, w1t, b1t, w2t, b2t, zft, l1wst, l1wat, l1bt, l2wst, l2wat, l2bt)
    return out_t


def kernel(x, edge_index, z, params):
    src = edge_index[0]
    dst = edge_index[1]
    c = jnp.zeros((_NN, _NN), jnp.float32).at[dst, src].add(1.0)
    w1t = params["gauge_W1"].reshape(_NN * _IN, _HID)
    b1t = params["gauge_b1"].reshape(_NN * _HID, 1)
    w2t = params["gauge_W2"].reshape(_NN * _HID, _OUT)
    b2t = params["gauge_b2"].T
    zft = (z @ params["z_W"] + params["z_b"]).reshape(_HID, 1)
    out_t = _run(
        c, x, w1t, b1t, w2t, b2t, zft,
        params["l1_Ws"], params["l1_Wa"], params["l1_b"].reshape(_HID, 1),
        params["l2_Ws"], params["l2_Wa"], params["l2_b"].reshape(_OUT, 1),
    )
    return out_t
